# baseline (speedup 1.0000x reference)
import functools

import jax
import jax.numpy as jnp
from jax import lax
from jax.experimental import pallas as pl
from jax.experimental.pallas import tpu as pltpu
from jax.experimental.pallas import tpu_sc as plsc

E = 8
TOPK_COEF = 0.02 * 8
L = 16
NUM_CORES = 2
NUM_SUBCORES = 16
NW = NUM_CORES * NUM_SUBCORES
T = 131072
CHUNK = T // NW
STEPS = CHUNK // L
MASK_N = 8192
NACC = 2 * E + 1


def _sc_body(gate_hbm, mask_hbm, out_hbm, chunk_v, mask_v, part_v):
    wid = lax.axis_index("c") * NUM_SUBCORES + lax.axis_index("s")
    base = wid * CHUNK
    pltpu.sync_copy(gate_hbm.at[pl.ds(base * E, CHUNK * E)], chunk_v)
    mbase = lax.rem(base, MASK_N)
    pltpu.sync_copy(mask_hbm.at[pl.ds(mbase, CHUNK)], mask_v)

    lanes = lax.iota(jnp.int32, L)
    zero_f = jnp.zeros((L,), jnp.float32)
    neg_big = jnp.full((L,), -3.0e38, jnp.float32)
    sentinel = jnp.full((L,), E, jnp.int32)
    idx_e = [jnp.full((L,), e, jnp.int32) for e in range(E)]

    def step(i, accs):
        tok = lanes + i * L
        flat = tok * E
        x = [plsc.load_gather(chunk_v, [flat + idx_e[e]]) for e in range(E)]
        mf = plsc.load_gather(mask_v, [tok]).astype(jnp.float32)
        m1 = x[0]
        for e in range(1, E):
            m1 = jnp.maximum(m1, x[e])
        s = [jnp.exp(x[e] - m1) for e in range(E)]
        den = s[0]
        for e in range(1, E):
            den = den + s[e]
        w = mf / den
        i1 = sentinel
        for e in range(E):
            i1 = jnp.minimum(i1, jnp.where(x[e] == m1, idx_e[e], sentinel))
        oh1 = [idx_e[e] == i1 for e in range(E)]
        x2 = [jnp.where(oh1[e], neg_big, x[e]) for e in range(E)]
        m2 = x2[0]
        for e in range(1, E):
            m2 = jnp.maximum(m2, x2[e])
        i2 = sentinel
        for e in range(E):
            i2 = jnp.minimum(i2, jnp.where(x2[e] == m2, idx_e[e], sentinel))
        new = []
        for e in range(E):
            ind = oh1[e] | (idx_e[e] == i2)
            new.append(accs[e] + jnp.where(ind, mf, zero_f))
        for e in range(E):
            new.append(accs[E + e] + s[e] * w)
        new.append(accs[2 * E] + mf)
        return tuple(new)

    init = tuple(jnp.zeros((L,), jnp.float32) for _ in range(NACC))
    accs = lax.fori_loop(0, STEPS, step, init)
    for j in range(NACC):
        part_v[j, :] = accs[j]
    pltpu.sync_copy(part_v, out_hbm.at[wid])


_sc_partials = functools.partial(
    pl.kernel,
    out_type=jax.ShapeDtypeStruct((NW, NACC, L), jnp.float32),
    mesh=plsc.VectorSubcoreMesh(
        core_axis_name="c", subcore_axis_name="s",
        num_cores=NUM_CORES, num_subcores=NUM_SUBCORES),
    compiler_params=pltpu.CompilerParams(needs_layout_passes=False),
    scratch_types=[
        pltpu.VMEM((CHUNK * E,), jnp.float32),
        pltpu.VMEM((CHUNK,), jnp.int32),
        pltpu.VMEM((NACC, L), jnp.float32),
    ],
)(_sc_body)


def _fin_body(x_ref, o_ref):
    x = x_ref[...]
    s = jnp.sum(x, axis=0)
    tot = jnp.sum(s, axis=1, keepdims=True)
    c = tot[0:E, :]
    p = tot[E:2 * E, :]
    m = tot[2 * E:, :]
    o_ref[...] = TOPK_COEF * jnp.sum(c * p, keepdims=True) / (m * m)


def kernel(gate_logits, attention_mask):
    mask_flat = attention_mask.reshape(-1)
    parts = _sc_partials(gate_logits.reshape(-1), mask_flat)
    out = pl.pallas_call(
        _fin_body,
        out_shape=jax.ShapeDtypeStruct((1, 1), jnp.float32),
    )(parts)
    return out[0, 0]


# speedup vs baseline: 3.4190x; 3.4190x over previous
import functools

import jax
import jax.numpy as jnp
from jax import lax
from jax.experimental import pallas as pl
from jax.experimental.pallas import tpu as pltpu
from jax.experimental.pallas import tpu_sc as plsc

E = 8
LOSS_COEF = 0.02 * 8
L = 16
LPT = 128
NUM_CORES = 2
NUM_SUBCORES = 16
NW = NUM_CORES * NUM_SUBCORES
T = 131072
NTILES = T // LPT
TILES_PER_W = NTILES // NW
CHUNK = T // NW
STEPS = CHUNK // L
SUBSTEPS = LPT // L
MASK_N = 8192
NACC = 2 * E + 1


def _sc_body(gate_hbm, mask_hbm, out_hbm, chunk_v, mask_v, part_v):
    wid = lax.axis_index("c") * NUM_SUBCORES + lax.axis_index("s")
    pltpu.sync_copy(gate_hbm.at[pl.ds(wid * TILES_PER_W, TILES_PER_W)], chunk_v)
    mbase = lax.rem(wid * CHUNK, MASK_N)
    pltpu.sync_copy(mask_hbm.at[pl.ds(mbase, CHUNK)], mask_v)

    zero_f = jnp.zeros((L,), jnp.float32)
    neg_big = jnp.full((L,), -3.0e38, jnp.float32)
    sentinel = jnp.full((L,), E, jnp.int32)
    idx_e = [jnp.full((L,), e, jnp.int32) for e in range(E)]

    def step(i, accs):
        jj = i // SUBSTEPS
        s = (i % SUBSTEPS) * L
        x = [chunk_v[jj, e, pl.ds(s, L)] for e in range(E)]
        mf = mask_v[pl.ds(i * L, L)].astype(jnp.float32)
        m1 = x[0]
        for e in range(1, E):
            m1 = jnp.maximum(m1, x[e])
        sm = [jnp.exp(x[e] - m1) for e in range(E)]
        den = sm[0]
        for e in range(1, E):
            den = den + sm[e]
        w = mf / den
        i1 = sentinel
        for e in range(E):
            i1 = jnp.minimum(i1, jnp.where(x[e] == m1, idx_e[e], sentinel))
        oh1 = [idx_e[e] == i1 for e in range(E)]
        x2 = [jnp.where(oh1[e], neg_big, x[e]) for e in range(E)]
        m2 = x2[0]
        for e in range(1, E):
            m2 = jnp.maximum(m2, x2[e])
        i2 = sentinel
        for e in range(E):
            i2 = jnp.minimum(i2, jnp.where(x2[e] == m2, idx_e[e], sentinel))
        new = []
        for e in range(E):
            ind = oh1[e] | (idx_e[e] == i2)
            new.append(accs[e] + jnp.where(ind, mf, zero_f))
        for e in range(E):
            new.append(accs[E + e] + sm[e] * w)
        new.append(accs[2 * E] + mf)
        return tuple(new)

    init = tuple(jnp.zeros((L,), jnp.float32) for _ in range(NACC))
    accs = lax.fori_loop(0, STEPS, step, init)
    for j in range(NACC):
        part_v[j, :] = accs[j]
    pltpu.sync_copy(part_v, out_hbm.at[wid])


_sc_partials = functools.partial(
    pl.kernel,
    out_type=jax.ShapeDtypeStruct((NW, NACC, L), jnp.float32),
    mesh=plsc.VectorSubcoreMesh(
        core_axis_name="c", subcore_axis_name="s",
        num_cores=NUM_CORES, num_subcores=NUM_SUBCORES),
    compiler_params=pltpu.CompilerParams(needs_layout_passes=False),
    scratch_types=[
        pltpu.VMEM((TILES_PER_W, E, LPT), jnp.float32),
        pltpu.VMEM((CHUNK,), jnp.int32),
        pltpu.VMEM((NACC, L), jnp.float32),
    ],
)(_sc_body)


def _fin_body(x_ref, o_ref):
    x = x_ref[...]
    s = jnp.sum(x, axis=0)
    tot = jnp.sum(s, axis=1, keepdims=True)
    c = tot[0:E, :]
    p = tot[E:2 * E, :]
    m = tot[2 * E:, :]
    o_ref[...] = LOSS_COEF * jnp.sum(c * p, keepdims=True) / (m * m)


def kernel(gate_logits, attention_mask):
    gate3d = gate_logits.T.reshape(E, NTILES, LPT).transpose(1, 0, 2)
    mask_flat = attention_mask.reshape(-1)
    parts = _sc_partials(gate3d, mask_flat)
    out = pl.pallas_call(
        _fin_body,
        out_shape=jax.ShapeDtypeStruct((1, 1), jnp.float32),
    )(parts)
    return out[0, 0]
